# baseline (speedup 1.0000x reference)
import functools

import jax
import jax.numpy as jnp
from jax import lax
from jax.experimental import pallas as pl
from jax.experimental.pallas import tpu as pltpu
from jax.experimental.pallas import tpu_sc as plsc

N = 10000
N_IN = 128
N_H = 64
N_IDX = 5000

_ROWS = 1000
_GRID = N // _ROWS

_NC = 2
_NS = 16
_NW = _NC * _NS
_LANES = 16
_CHUNK = 160
_LAST = N_IDX - (_NW - 1) * _CHUNK


def _norm_body(x_ref, w1_ref, b1_ref, w2_ref, b2_ref, ones_ref, out_ref):
    x = x_ref[...]
    h = jnp.dot(x, w1_ref[...], preferred_element_type=jnp.float32) + b1_ref[...]
    h = jnp.maximum(h, 0.0)
    xr = jnp.dot(h, w2_ref[...], preferred_element_type=jnp.float32) + b2_ref[...]
    d = x - xr
    s = jnp.dot(d * d, ones_ref[...], preferred_element_type=jnp.float32)
    out_ref[...] = jnp.sqrt(s)


def _row_norms(x, w1, b1, w2, b2):
    ones = jnp.ones((N_IN, 1), dtype=jnp.float32)
    return pl.pallas_call(
        _norm_body,
        grid=(_GRID,),
        in_specs=[
            pl.BlockSpec((_ROWS, N_IN), lambda i: (i, 0)),
            pl.BlockSpec((N_IN, N_H), lambda i: (0, 0)),
            pl.BlockSpec((1, N_H), lambda i: (0, 0)),
            pl.BlockSpec((N_H, N_IN), lambda i: (0, 0)),
            pl.BlockSpec((1, N_IN), lambda i: (0, 0)),
            pl.BlockSpec((N_IN, 1), lambda i: (0, 0)),
        ],
        out_specs=pl.BlockSpec((_ROWS, 1), lambda i: (i, 0)),
        out_shape=jax.ShapeDtypeStruct((N, 1), jnp.float32),
    )(x, w1, b1.reshape(1, N_H), w2, b2.reshape(1, N_IN), ones)


def _sc_body(nrm_hbm, idx_tr_hbm, idx_te_hbm, te_out, part_out,
             idx_a, val_a, idx_b, val_b, acc_v, sem_a, sem_b):
    wid = lax.axis_index("s") * _NC + lax.axis_index("c")
    base = wid * _CHUNK
    lanes = lax.iota(jnp.int32, _LANES)

    @pl.when(wid < _NW - 1)
    def _full():
        pltpu.sync_copy(idx_te_hbm.at[pl.ds(base, _CHUNK)], idx_a)
        pltpu.sync_copy(idx_tr_hbm.at[pl.ds(base, _CHUNK)], idx_b)
        cp_a = pltpu.async_copy(nrm_hbm.at[idx_a], val_a, sem_a)
        cp_b = pltpu.async_copy(nrm_hbm.at[idx_b], val_b, sem_b)
        cp_a.wait()
        pltpu.sync_copy(val_a, te_out.at[pl.ds(base, _CHUNK)])
        cp_b.wait()
        acc = jnp.zeros((_LANES,), jnp.float32)
        for j in range(_CHUNK // _LANES):
            acc = acc + val_b[pl.ds(j * _LANES, _LANES)]
        acc_v[...] = acc
        pltpu.sync_copy(acc_v, part_out.at[wid])

    @pl.when(wid == _NW - 1)
    def _ragged():
        pltpu.sync_copy(idx_te_hbm.at[pl.ds(base, _LAST)], idx_a.at[pl.ds(0, _LAST)])
        pltpu.sync_copy(idx_tr_hbm.at[pl.ds(base, _LAST)], idx_b.at[pl.ds(0, _LAST)])
        cp_a = pltpu.async_copy(nrm_hbm.at[idx_a.at[pl.ds(0, _LAST)]],
                                val_a.at[pl.ds(0, _LAST)], sem_a)
        cp_b = pltpu.async_copy(nrm_hbm.at[idx_b.at[pl.ds(0, _LAST)]],
                                val_b.at[pl.ds(0, _LAST)], sem_b)
        cp_a.wait()
        pltpu.sync_copy(val_a.at[pl.ds(0, _LAST)], te_out.at[pl.ds(base, _LAST)])
        cp_b.wait()
        acc = jnp.zeros((_LANES,), jnp.float32)
        for j in range(_LAST // _LANES + 1):
            g = lanes + (base + j * _LANES)
            v = val_b[pl.ds(j * _LANES, _LANES)]
            acc = acc + jnp.where(g < N_IDX, v, 0.0)
        acc_v[...] = acc
        pltpu.sync_copy(acc_v, part_out.at[wid])


def _sc_gather(nrm, idx_tr, idx_te):
    mesh = plsc.VectorSubcoreMesh(core_axis_name="c", subcore_axis_name="s")
    run = functools.partial(
        pl.kernel,
        mesh=mesh,
        out_type=[
            jax.ShapeDtypeStruct((N_IDX,), jnp.float32),
            jax.ShapeDtypeStruct((_NW, _LANES), jnp.float32),
        ],
        scratch_types=[
            pltpu.VMEM((_CHUNK,), jnp.int32),
            pltpu.VMEM((_CHUNK,), jnp.float32),
            pltpu.VMEM((_CHUNK,), jnp.int32),
            pltpu.VMEM((_CHUNK,), jnp.float32),
            pltpu.VMEM((_LANES,), jnp.float32),
            pltpu.SemaphoreType.DMA,
            pltpu.SemaphoreType.DMA,
        ],
    )(_sc_body)
    return run(nrm, idx_tr, idx_te)


def kernel(seq1, adj, idx_train, idx_test, W_stru, b_stru,
           W_attr1, b_attr1, W_attr2, b_attr2):
    del adj, W_stru, b_stru
    nrm = _row_norms(seq1, W_attr1, b_attr1, W_attr2, b_attr2).reshape(N)
    te, parts = _sc_gather(nrm, idx_train.astype(jnp.int32),
                           idx_test.astype(jnp.int32))
    loss = jnp.sum(parts) * (1.0 / N_IDX)
    return (loss, te)


# speedup vs baseline: 1.4274x; 1.4274x over previous
import functools

import jax
import jax.numpy as jnp
from jax import lax
from jax.experimental import pallas as pl
from jax.experimental.pallas import tpu as pltpu
from jax.experimental.pallas import tpu_sc as plsc

N = 10000
N_IN = 128
N_H = 64
N_IDX = 5000

_ROWS = 1000
_GRID = N // _ROWS

_NC = 2
_NS = 16
_NW = _NC * _NS
_LANES = 16
_CHUNK = 160
_LAST = N_IDX - (_NW - 1) * _CHUNK


def _norm_body(x_ref, w1_ref, b1_ref, w2_ref, b2_ref, out_ref):
    x = x_ref[...]
    h = jnp.dot(x, w1_ref[...], preferred_element_type=jnp.float32) + b1_ref[...]
    h = jnp.maximum(h, 0.0)
    xr = jnp.dot(h, w2_ref[...], preferred_element_type=jnp.float32) + b2_ref[...]
    d = x - xr
    ones = jnp.ones((1, N_IN), dtype=jnp.float32)
    s = jax.lax.dot_general(ones, d * d, (((1,), (1,)), ((), ())),
                            preferred_element_type=jnp.float32)
    out_ref[...] = jnp.sqrt(s.reshape(N))


def _row_norms(x, w1, b1, w2, b2):
    return pl.pallas_call(
        _norm_body,
        out_shape=jax.ShapeDtypeStruct((N,), jnp.float32),
    )(x, w1, b1.reshape(1, N_H), w2, b2.reshape(1, N_IN))


def _sc_body(nrm_hbm, idx_tr_hbm, idx_te_hbm, te_out, part_out,
             idx_a, val_a, idx_b, val_b, acc_v, sem_a, sem_b):
    wid = lax.axis_index("s") * _NC + lax.axis_index("c")
    base = wid * _CHUNK
    lanes = lax.iota(jnp.int32, _LANES)

    @pl.when(wid < _NW - 1)
    def _full():
        pltpu.sync_copy(idx_te_hbm.at[pl.ds(base, _CHUNK)], idx_a)
        pltpu.sync_copy(idx_tr_hbm.at[pl.ds(base, _CHUNK)], idx_b)
        cp_a = pltpu.async_copy(nrm_hbm.at[idx_a], val_a, sem_a)
        cp_b = pltpu.async_copy(nrm_hbm.at[idx_b], val_b, sem_b)
        cp_a.wait()
        pltpu.sync_copy(val_a, te_out.at[pl.ds(base, _CHUNK)])
        cp_b.wait()
        acc = jnp.zeros((_LANES,), jnp.float32)
        for j in range(_CHUNK // _LANES):
            acc = acc + val_b[pl.ds(j * _LANES, _LANES)]
        acc_v[...] = acc
        pltpu.sync_copy(acc_v, part_out.at[wid])

    @pl.when(wid == _NW - 1)
    def _ragged():
        pltpu.sync_copy(idx_te_hbm.at[pl.ds(base, _LAST)], idx_a.at[pl.ds(0, _LAST)])
        pltpu.sync_copy(idx_tr_hbm.at[pl.ds(base, _LAST)], idx_b.at[pl.ds(0, _LAST)])
        cp_a = pltpu.async_copy(nrm_hbm.at[idx_a.at[pl.ds(0, _LAST)]],
                                val_a.at[pl.ds(0, _LAST)], sem_a)
        cp_b = pltpu.async_copy(nrm_hbm.at[idx_b.at[pl.ds(0, _LAST)]],
                                val_b.at[pl.ds(0, _LAST)], sem_b)
        cp_a.wait()
        pltpu.sync_copy(val_a.at[pl.ds(0, _LAST)], te_out.at[pl.ds(base, _LAST)])
        cp_b.wait()
        acc = jnp.zeros((_LANES,), jnp.float32)
        for j in range(_LAST // _LANES + 1):
            g = lanes + (base + j * _LANES)
            v = val_b[pl.ds(j * _LANES, _LANES)]
            acc = acc + jnp.where(g < N_IDX, v, 0.0)
        acc_v[...] = acc
        pltpu.sync_copy(acc_v, part_out.at[wid])


def _sc_gather(nrm, idx_tr, idx_te):
    mesh = plsc.VectorSubcoreMesh(core_axis_name="c", subcore_axis_name="s")
    run = functools.partial(
        pl.kernel,
        mesh=mesh,
        out_type=[
            jax.ShapeDtypeStruct((N_IDX,), jnp.float32),
            jax.ShapeDtypeStruct((_NW, _LANES), jnp.float32),
        ],
        scratch_types=[
            pltpu.VMEM((_CHUNK,), jnp.int32),
            pltpu.VMEM((_CHUNK,), jnp.float32),
            pltpu.VMEM((_CHUNK,), jnp.int32),
            pltpu.VMEM((_CHUNK,), jnp.float32),
            pltpu.VMEM((_LANES,), jnp.float32),
            pltpu.SemaphoreType.DMA,
            pltpu.SemaphoreType.DMA,
        ],
    )(_sc_body)
    return run(nrm, idx_tr, idx_te)


def kernel(seq1, adj, idx_train, idx_test, W_stru, b_stru,
           W_attr1, b_attr1, W_attr2, b_attr2):
    del adj, W_stru, b_stru
    nrm = _row_norms(seq1, W_attr1, b_attr1, W_attr2, b_attr2).reshape(N)
    te, parts = _sc_gather(nrm, idx_train.astype(jnp.int32),
                           idx_test.astype(jnp.int32))
    loss = jnp.sum(parts) * (1.0 / N_IDX)
    return (loss, te)
